# baseline (speedup 1.0000x reference)
import jax
import jax.numpy as jnp
from jax import lax
from jax.experimental import pallas as pl
from jax.experimental.pallas import tpu as pltpu
from jax.experimental.pallas import tpu_sc as plsc

_L = 6
_F = 2
_P = 131072
_NC = 2
_NS = 16
_NW = _NC * _NS
_PPW = _P // _NW
_BATCH = 16
_NROWS = 33832


def _body(in_hbm, out_hbm, gridv, xv, outv, sem):
    wid = lax.axis_index("s") * _NC + lax.axis_index("c")
    base = wid * _PPW

    cg = pltpu.async_copy(in_hbm.at[pl.ds(0, _NROWS)], gridv, sem)
    cxs = [pltpu.async_copy(in_hbm.at[pl.ds(_NROWS + r * _P + base, _PPW)],
                            xv.at[pl.ds(r * _PPW, _PPW)], sem)
           for r in range(3)]
    cg.wait()
    for c in cxs:
        c.wait()

    def batch(b):
        o = b * _BATCH
        x1 = xv[pl.ds(o, _BATCH)] + 1.0
        y1 = xv[pl.ds(_PPW + o, _BATCH)] + 1.0
        z1 = xv[pl.ds(2 * _PPW + o, _BATCH)] + 1.0
        gx5 = x1 * (2.0 ** (_L - 2))
        gy5 = y1 * (2.0 ** (_L - 2))
        gz5 = z1 * (2.0 ** (_L - 2))
        ix5 = gx5.astype(jnp.int32)
        iy5 = gy5.astype(jnp.int32)
        iz5 = gz5.astype(jnp.int32)
        for l in range(_L):
            vs = 2 ** l
            sh = _L - 1 - l
            scale = 0.5 ** sh
            gx = gx5 * scale if sh else gx5
            gy = gy5 * scale if sh else gy5
            gz = gz5 * scale if sh else gz5
            ix = lax.shift_right_logical(ix5, sh) if sh else ix5
            iy = lax.shift_right_logical(iy5, sh) if sh else iy5
            iz = lax.shift_right_logical(iz5, sh) if sh else iz5
            fx = plsc.pack(gx - ix.astype(jnp.float32),
                           gx - ix.astype(jnp.float32),
                           format=plsc.PackFormat.INTERLEAVED)
            fy = plsc.pack(gy - iy.astype(jnp.float32),
                           gy - iy.astype(jnp.float32),
                           format=plsc.PackFormat.INTERLEAVED)
            fz = plsc.pack(gz - iz.astype(jnp.float32),
                           gz - iz.astype(jnp.float32),
                           format=plsc.PackFormat.INTERLEAVED)
            i000 = ix + iy * vs + iz * (vs * vs)
            q = []
            for dz in (0, vs * vs):
                for dy in (0, vs):
                    for dx in (0, 1):
                        i = i000 + (dx + dy + dz)
                        q.append(plsc.bitcast(plsc.load_gather(gridv, [i]),
                                              jnp.bfloat16))
            a0 = q[0] + fx * (q[1] - q[0])
            a1 = q[2] + fx * (q[3] - q[2])
            a2 = q[4] + fx * (q[5] - q[4])
            a3 = q[6] + fx * (q[7] - q[6])
            b0 = a0 + fy * (a1 - a0)
            b1 = a2 + fy * (a3 - a2)
            r = b0 + fz * (b1 - b0)
            r0, r1 = plsc.unpack(r, format=plsc.PackFormat.INTERLEAVED)
            outv[pl.ds((l * _F + 0) * _PPW + o, _BATCH)] = r0
            outv[pl.ds((l * _F + 1) * _PPW + o, _BATCH)] = r1

    nch = 4
    ch = _PPW // (nch * _BATCH)
    cw = _PPW // nch
    cos = []
    for k in range(nch):
        plsc.parallel_loop(k * ch, (k + 1) * ch, unroll=8)(batch)
        cos.append([pltpu.async_copy(
            outv.at[pl.ds(r * _PPW + k * cw, cw)],
            out_hbm.at[pl.ds(r * _P + base + k * cw, cw)], sem)
            for r in range(_L * _F)])
    for cs in cos:
        for c in cs:
            c.wait()


@jax.jit
def kernel(x, grid):
    xt = x.T.reshape(-1)
    grid_pairs = lax.bitcast_convert_type(
        grid[:_NROWS].astype(jnp.bfloat16), jnp.float32)
    packed = jnp.concatenate([grid_pairs, xt])
    mesh = plsc.VectorSubcoreMesh(core_axis_name="c", subcore_axis_name="s")
    out = pl.kernel(
        _body,
        out_type=jax.ShapeDtypeStruct((_L * _F * _P,), jnp.float32),
        mesh=mesh,
        scratch_types=[
            pltpu.VMEM((_NROWS,), jnp.float32),
            pltpu.VMEM((3 * _PPW,), jnp.float32),
            pltpu.VMEM((_L * _F * _PPW,), jnp.float32),
            pltpu.SemaphoreType.DMA,
        ],
        compiler_params=pltpu.CompilerParams(needs_layout_passes=False),
    )(packed)
    return out.reshape(_L * _F, _P).T


# speedup vs baseline: 1.0435x; 1.0435x over previous
import jax
import jax.numpy as jnp
from jax import lax
from jax.experimental import pallas as pl
from jax.experimental.pallas import tpu as pltpu
from jax.experimental.pallas import tpu_sc as plsc

_L = 6
_F = 2
_P = 131072
_NC = 2
_NS = 16
_NW = _NC * _NS
_PPW = _P // _NW
_BATCH = 16
_NROWS = 33832


def _body(in_hbm, out_hbm, gridv, xv, outv, sem):
    wid = lax.axis_index("s") * _NC + lax.axis_index("c")
    base = wid * _PPW

    cg = pltpu.async_copy(in_hbm.at[pl.ds(0, _NROWS)], gridv, sem)
    cxs = [pltpu.async_copy(in_hbm.at[pl.ds(_NROWS + r * _P + base, _PPW)],
                            xv.at[pl.ds(r * _PPW, _PPW)], sem)
           for r in range(3)]
    cg.wait()
    for c in cxs:
        c.wait()

    def batch(b):
        o = b * _BATCH
        x1 = xv[pl.ds(o, _BATCH)] + 1.0
        y1 = xv[pl.ds(_PPW + o, _BATCH)] + 1.0
        z1 = xv[pl.ds(2 * _PPW + o, _BATCH)] + 1.0
        gx5 = x1 * (2.0 ** (_L - 2))
        gy5 = y1 * (2.0 ** (_L - 2))
        gz5 = z1 * (2.0 ** (_L - 2))
        ix5 = gx5.astype(jnp.int32)
        iy5 = gy5.astype(jnp.int32)
        iz5 = gz5.astype(jnp.int32)
        for l in range(_L):
            vs = 2 ** l
            sh = _L - 1 - l
            scale = 0.5 ** sh
            gx = gx5 * scale if sh else gx5
            gy = gy5 * scale if sh else gy5
            gz = gz5 * scale if sh else gz5
            ix = lax.shift_right_logical(ix5, sh) if sh else ix5
            iy = lax.shift_right_logical(iy5, sh) if sh else iy5
            iz = lax.shift_right_logical(iz5, sh) if sh else iz5
            fx = plsc.pack(gx - ix.astype(jnp.float32),
                           gx - ix.astype(jnp.float32),
                           format=plsc.PackFormat.INTERLEAVED)
            fy = plsc.pack(gy - iy.astype(jnp.float32),
                           gy - iy.astype(jnp.float32),
                           format=plsc.PackFormat.INTERLEAVED)
            fz = plsc.pack(gz - iz.astype(jnp.float32),
                           gz - iz.astype(jnp.float32),
                           format=plsc.PackFormat.INTERLEAVED)
            i000 = ix + iy * vs + iz * (vs * vs)
            q = []
            for dz in (0, vs * vs):
                for dy in (0, vs):
                    for dx in (0, 1):
                        i = i000 + (dx + dy + dz)
                        q.append(plsc.bitcast(plsc.load_gather(gridv, [i]),
                                              jnp.bfloat16))
            a0 = q[0] + fx * (q[1] - q[0])
            a1 = q[2] + fx * (q[3] - q[2])
            a2 = q[4] + fx * (q[5] - q[4])
            a3 = q[6] + fx * (q[7] - q[6])
            b0 = a0 + fy * (a1 - a0)
            b1 = a2 + fy * (a3 - a2)
            r = b0 + fz * (b1 - b0)
            r0, r1 = plsc.unpack(r, format=plsc.PackFormat.INTERLEAVED)
            outv[pl.ds((l * _F + 0) * _PPW + o, _BATCH)] = r0
            outv[pl.ds((l * _F + 1) * _PPW + o, _BATCH)] = r1

    half = _PPW // (2 * _BATCH)
    hw = _PPW // 2
    plsc.parallel_loop(0, half, unroll=8)(batch)
    cos = [pltpu.async_copy(outv.at[pl.ds(r * _PPW, hw)],
                            out_hbm.at[pl.ds(r * _P + base, hw)], sem)
           for r in range(_L * _F)]
    plsc.parallel_loop(half, 2 * half, unroll=8)(batch)
    for c in cos:
        c.wait()
    for r in range(_L * _F):
        pltpu.sync_copy(outv.at[pl.ds(r * _PPW + hw, hw)],
                        out_hbm.at[pl.ds(r * _P + base + hw, hw)])


@jax.jit
def kernel(x, grid):
    xt = x.T.reshape(-1)
    grid_pairs = lax.bitcast_convert_type(
        grid[:_NROWS].astype(jnp.bfloat16), jnp.float32)
    packed = jnp.concatenate([grid_pairs, xt])
    mesh = plsc.VectorSubcoreMesh(core_axis_name="c", subcore_axis_name="s")
    out = pl.kernel(
        _body,
        out_type=jax.ShapeDtypeStruct((_L * _F * _P,), jnp.float32),
        mesh=mesh,
        scratch_types=[
            pltpu.VMEM((_NROWS,), jnp.float32),
            pltpu.VMEM((3 * _PPW,), jnp.float32),
            pltpu.VMEM((_L * _F * _PPW,), jnp.float32),
            pltpu.SemaphoreType.DMA,
        ],
        compiler_params=pltpu.CompilerParams(needs_layout_passes=False),
    )(packed)
    return out.reshape(_L * _F, _P).T
